# baseline (speedup 1.0000x reference)
import jax
import jax.numpy as jnp
from jax import lax
from jax.experimental import pallas as pl
from jax.experimental.pallas import tpu as pltpu
from jax.experimental.pallas import tpu_sc as plsc

NUM_EMB = 1000000
DIM = 64
PDIM = 128
BATCH = 4096
HIST = 50

_info = plsc.get_sparse_core_info()
NC, NS = _info.num_cores, _info.num_subcores
NW = NC * NS
TOTAL = BATCH * HIST
PER_W = TOTAL // NW
U = 8
GROUP = U * 16
NGROUP = PER_W // GROUP
NBUF = 4


def _gather_kernel(table_hbm, idx_hbm, out_hbm, idx_v, rows_v, sem_g, sem_o):
    wid = lax.axis_index("s") * NC + lax.axis_index("c")
    base = wid * PER_W
    pltpu.sync_copy(idx_hbm.at[pl.ds(base, PER_W)], idx_v)

    def fire(g, b):
        for u in range(U):
            vec = idx_v[pl.ds(g * GROUP + u * 16, 16)]
            pltpu.make_async_copy(
                table_hbm.at[vec],
                rows_v.at[pl.ds(b * GROUP + u * 16, 16)],
                sem_g.at[b],
            ).start()

    def drain_gather(b):
        pltpu.make_async_copy(
            table_hbm.at[pl.ds(0, GROUP)],
            rows_v.at[pl.ds(b * GROUP, GROUP)],
            sem_g.at[b],
        ).wait()

    def out_start(g, b):
        pltpu.make_async_copy(
            rows_v.at[pl.ds(b * GROUP, GROUP)],
            out_hbm.at[pl.ds(base + g * GROUP, GROUP)],
            sem_o.at[b],
        ).start()

    def out_drain(b):
        pltpu.make_async_copy(
            rows_v.at[pl.ds(b * GROUP, GROUP)],
            out_hbm.at[pl.ds(base, GROUP)],
            sem_o.at[b],
        ).wait()

    fire(0, 0)

    def body(g, carry):
        b = lax.rem(g, NBUF)
        gn = g + 1
        bn = lax.rem(gn, NBUF)

        @pl.when(gn < NGROUP)
        def _():
            @pl.when(gn >= NBUF)
            def _():
                out_drain(bn)

            fire(gn, bn)

        drain_gather(b)
        out_start(g, b)
        return carry

    lax.fori_loop(0, NGROUP, body, 0, unroll=False)

    for b in range(min(NBUF, NGROUP)):
        out_drain(b)


@jax.jit
def kernel(token_ids, weights):
    idx = token_ids.astype(jnp.int32).reshape(TOTAL)
    wtab = jnp.pad(weights, ((0, 0), (0, PDIM - DIM)))
    mesh = plsc.VectorSubcoreMesh(core_axis_name="c", subcore_axis_name="s")
    out = pl.kernel(
        _gather_kernel,
        out_type=jax.ShapeDtypeStruct((TOTAL, PDIM), jnp.float32),
        mesh=mesh,
        scratch_types=[
            pltpu.VMEM((PER_W,), jnp.int32),
            pltpu.VMEM((NBUF * GROUP, PDIM), jnp.float32),
            pltpu.SemaphoreType.DMA((NBUF,)),
            pltpu.SemaphoreType.DMA((NBUF,)),
        ],
    )(wtab, idx)
    return out[:, :DIM].reshape(BATCH, HIST, DIM)


# speedup vs baseline: 1.0023x; 1.0023x over previous
import jax
import jax.numpy as jnp
from jax import lax
from jax.experimental import pallas as pl
from jax.experimental.pallas import tpu as pltpu
from jax.experimental.pallas import tpu_sc as plsc

NUM_EMB = 1000000
DIM = 64
PDIM = 128
BATCH = 4096
HIST = 50

_info = plsc.get_sparse_core_info()
NC, NS = _info.num_cores, _info.num_subcores
NW = NC * NS
TOTAL = BATCH * HIST
PER_W = TOTAL // NW
U = 8
GROUP = U * 16
NGROUP = PER_W // GROUP
NBUF = 4


def _gather_kernel(table_hbm, idx_hbm, out_hbm, idx_v, rows_v, sem_g, sem_o):
    wid = lax.axis_index("s") * NC + lax.axis_index("c")
    base = wid * PER_W
    pltpu.sync_copy(idx_hbm.at[pl.ds(base, PER_W)], idx_v)

    def fire(g, b):
        def batch(u, c):
            vec = idx_v[pl.ds(g * GROUP + u * 16, 16)]
            for l in range(16):
                tok = vec[l]
                pltpu.make_async_copy(
                    table_hbm.at[pl.ds(tok, 1)],
                    rows_v.at[pl.ds(b * GROUP + u * 16 + l, 1)],
                    sem_g.at[b],
                ).start()
            return c

        lax.fori_loop(0, U, batch, 0, unroll=False)

    def drain_gather(b):
        pltpu.make_async_copy(
            table_hbm.at[pl.ds(0, GROUP)],
            rows_v.at[pl.ds(b * GROUP, GROUP)],
            sem_g.at[b],
        ).wait()

    def out_start(g, b):
        pltpu.make_async_copy(
            rows_v.at[pl.ds(b * GROUP, GROUP)],
            out_hbm.at[pl.ds(base + g * GROUP, GROUP)],
            sem_o.at[b],
        ).start()

    def out_drain(b):
        pltpu.make_async_copy(
            rows_v.at[pl.ds(b * GROUP, GROUP)],
            out_hbm.at[pl.ds(base, GROUP)],
            sem_o.at[b],
        ).wait()

    fire(0, 0)

    def body(g, carry):
        b = lax.rem(g, NBUF)
        gn = g + 1
        bn = lax.rem(gn, NBUF)

        @pl.when(gn < NGROUP)
        def _():
            @pl.when(gn >= NBUF)
            def _():
                out_drain(bn)

            fire(gn, bn)

        drain_gather(b)
        out_start(g, b)
        return carry

    lax.fori_loop(0, NGROUP, body, 0, unroll=False)

    for b in range(min(NBUF, NGROUP)):
        out_drain(b)


@jax.jit
def kernel(token_ids, weights):
    idx = token_ids.astype(jnp.int32).reshape(TOTAL)
    wtab = jnp.pad(weights, ((0, 0), (0, PDIM - DIM)))
    mesh = plsc.VectorSubcoreMesh(core_axis_name="c", subcore_axis_name="s")
    out = pl.kernel(
        _gather_kernel,
        out_type=jax.ShapeDtypeStruct((TOTAL, PDIM), jnp.float32),
        mesh=mesh,
        scratch_types=[
            pltpu.VMEM((PER_W,), jnp.int32),
            pltpu.VMEM((NBUF * GROUP, PDIM), jnp.float32),
            pltpu.SemaphoreType.DMA((NBUF,)),
            pltpu.SemaphoreType.DMA((NBUF,)),
        ],
    )(wtab, idx)
    return out[:, :DIM].reshape(BATCH, HIST, DIM)


# speedup vs baseline: 1.3285x; 1.3254x over previous
import jax
import jax.numpy as jnp
from jax import lax
from jax.experimental import pallas as pl
from jax.experimental.pallas import tpu as pltpu
from jax.experimental.pallas import tpu_sc as plsc

NUM_EMB = 1000000
DIM = 64
BATCH = 4096
HIST = 50

_info = plsc.get_sparse_core_info()
NC, NS = _info.num_cores, _info.num_subcores
NW = NC * NS
TOTAL = BATCH * HIST
PER_W = TOTAL // NW
U = 8
GROUP = U * 16
NGROUP = PER_W // GROUP
NBUF = 4


def _gather_kernel(table_hbm, idx_hbm, out_hbm, idx_v, rows_v, sem_g, sem_o):
    wid = lax.axis_index("s") * NC + lax.axis_index("c")
    base = wid * PER_W
    pltpu.sync_copy(idx_hbm.at[pl.ds(base, PER_W)], idx_v)

    def fire(g, b):
        def batch(u, c):
            vec = idx_v[pl.ds(g * GROUP + u * 16, 16)]
            for l in range(16):
                tok = vec[l]
                pltpu.make_async_copy(
                    table_hbm.at[pl.ds(tok, 1)],
                    rows_v.at[pl.ds(b * GROUP + u * 16 + l, 1)],
                    sem_g.at[b],
                ).start()
            return c

        lax.fori_loop(0, U, batch, 0, unroll=False)

    def drain_gather(b):
        pltpu.make_async_copy(
            table_hbm.at[pl.ds(0, GROUP)],
            rows_v.at[pl.ds(b * GROUP, GROUP)],
            sem_g.at[b],
        ).wait()

    def out_start(g, b):
        pltpu.make_async_copy(
            rows_v.at[pl.ds(b * GROUP, GROUP)],
            out_hbm.at[pl.ds(base + g * GROUP, GROUP)],
            sem_o.at[b],
        ).start()

    def out_drain(b):
        pltpu.make_async_copy(
            rows_v.at[pl.ds(b * GROUP, GROUP)],
            out_hbm.at[pl.ds(base, GROUP)],
            sem_o.at[b],
        ).wait()

    fire(0, 0)

    def body(g, carry):
        b = lax.rem(g, NBUF)
        gn = g + 1
        bn = lax.rem(gn, NBUF)

        @pl.when(gn < NGROUP)
        def _():
            @pl.when(gn >= NBUF)
            def _():
                out_drain(bn)

            fire(gn, bn)

        drain_gather(b)
        out_start(g, b)
        return carry

    lax.fori_loop(0, NGROUP, body, 0, unroll=False)

    for b in range(min(NBUF, NGROUP)):
        out_drain(b)


@jax.jit
def kernel(token_ids, weights):
    idx = token_ids.astype(jnp.int32).reshape(TOTAL)
    mesh = plsc.VectorSubcoreMesh(core_axis_name="c", subcore_axis_name="s")
    out = pl.kernel(
        _gather_kernel,
        out_type=jax.ShapeDtypeStruct((TOTAL, DIM), jnp.float32),
        mesh=mesh,
        scratch_types=[
            pltpu.VMEM((PER_W,), jnp.int32),
            pltpu.VMEM((NBUF * GROUP, DIM), jnp.float32),
            pltpu.SemaphoreType.DMA((NBUF,)),
            pltpu.SemaphoreType.DMA((NBUF,)),
        ],
    )(weights, idx)
    return out.reshape(BATCH, HIST, DIM)


# speedup vs baseline: 1.5147x; 1.1401x over previous
import jax
import jax.numpy as jnp
from jax import lax
from jax.experimental import pallas as pl
from jax.experimental.pallas import tpu as pltpu
from jax.experimental.pallas import tpu_sc as plsc

NUM_EMB = 1000000
DIM = 64
BATCH = 4096
HIST = 50

_info = plsc.get_sparse_core_info()
NC, NS = _info.num_cores, _info.num_subcores
NW = NC * NS
TOTAL = BATCH * HIST
PER_W = TOTAL // NW
B_PER_W = BATCH // NW
GB = 8
GROUP = GB * HIST
NGROUP = B_PER_W // GB
NBUF = 2


def _gather_kernel(table_hbm, idx_hbm, out_hbm, idx_v, rows_v, sem_g, sem_o):
    wid = lax.axis_index("s") * NC + lax.axis_index("c")
    base = wid * PER_W
    bbase = wid * B_PER_W
    pltpu.sync_copy(idx_hbm.at[pl.ds(base, PER_W)], idx_v)

    def fire(g, b):
        def batch16(u, c):
            vec = idx_v[pl.ds(g * GROUP + u * 16, 16)]
            for l in range(16):
                j = u * 16 + l
                tok = vec[l]
                pltpu.make_async_copy(
                    table_hbm.at[pl.ds(tok, 1)],
                    rows_v.at[b, lax.div(j, HIST), pl.ds(lax.rem(j, HIST), 1)],
                    sem_g.at[b],
                ).start()
            return c

        lax.fori_loop(0, GROUP // 16, batch16, 0, unroll=False)

    def drain_gather(b):
        pltpu.make_async_copy(
            out_hbm.at[pl.ds(bbase, GB)], rows_v.at[b], sem_g.at[b]
        ).wait()

    def out_start(g, b):
        pltpu.make_async_copy(
            rows_v.at[b], out_hbm.at[pl.ds(bbase + g * GB, GB)], sem_o.at[b]
        ).start()

    def out_drain(b):
        pltpu.make_async_copy(
            rows_v.at[b], out_hbm.at[pl.ds(bbase, GB)], sem_o.at[b]
        ).wait()

    fire(0, 0)

    def body(g, carry):
        b = lax.rem(g, NBUF)
        gn = g + 1
        bn = lax.rem(gn, NBUF)

        @pl.when(gn < NGROUP)
        def _():
            @pl.when(gn >= NBUF)
            def _():
                out_drain(bn)

            fire(gn, bn)

        drain_gather(b)
        out_start(g, b)
        return carry

    lax.fori_loop(0, NGROUP, body, 0, unroll=False)

    for b in range(min(NBUF, NGROUP)):
        out_drain(b)


@jax.jit
def kernel(token_ids, weights):
    idx = token_ids.astype(jnp.int32).reshape(TOTAL)
    mesh = plsc.VectorSubcoreMesh(core_axis_name="c", subcore_axis_name="s")
    out = pl.kernel(
        _gather_kernel,
        out_type=jax.ShapeDtypeStruct((BATCH, HIST, DIM), jnp.float32),
        mesh=mesh,
        scratch_types=[
            pltpu.VMEM((PER_W,), jnp.int32),
            pltpu.VMEM((NBUF, GB, HIST, DIM), jnp.float32),
            pltpu.SemaphoreType.DMA((NBUF,)),
            pltpu.SemaphoreType.DMA((NBUF,)),
        ],
    )(weights, idx)
    return out


# speedup vs baseline: 1.5182x; 1.0024x over previous
import jax
import jax.numpy as jnp
from jax import lax
from jax.experimental import pallas as pl
from jax.experimental.pallas import tpu as pltpu
from jax.experimental.pallas import tpu_sc as plsc

NUM_EMB = 1000000
DIM = 64
BATCH = 4096
HIST = 50

_info = plsc.get_sparse_core_info()
NC, NS = _info.num_cores, _info.num_subcores
NW = NC * NS
TOTAL = BATCH * HIST
PER_W = TOTAL // NW
B_PER_W = BATCH // NW
GB = 8
GROUP = GB * HIST
NGROUP = B_PER_W // GB
NBUF = 2
HPAD = 64


def _gather_kernel(table_hbm, idx_hbm, out_hbm, idx_v, rows_v, sem_g, sem_o):
    wid = lax.axis_index("s") * NC + lax.axis_index("c")
    bbase = wid * B_PER_W
    pltpu.sync_copy(idx_hbm.at[pl.ds(bbase * HPAD, B_PER_W * HPAD)], idx_v)

    def fire(g, b):
        def batchp(p, c):
            for hc in range(4):
                vec = idx_v[pl.ds((g * GB + p) * HPAD + hc * 16, 16)]
                for l in range(16 if hc < 3 else HIST - 48):
                    tok = vec[l]
                    pltpu.make_async_copy(
                        table_hbm.at[pl.ds(tok, 1)],
                        rows_v.at[b, p, pl.ds(hc * 16 + l, 1)],
                        sem_g.at[b],
                    ).start()
            return c

        lax.fori_loop(0, GB, batchp, 0, unroll=False)

    def drain_gather(b):
        pltpu.make_async_copy(
            out_hbm.at[pl.ds(bbase, GB)], rows_v.at[b], sem_g.at[b]
        ).wait()

    def out_start(g, b):
        pltpu.make_async_copy(
            rows_v.at[b], out_hbm.at[pl.ds(bbase + g * GB, GB)], sem_o.at[b]
        ).start()

    def out_drain(b):
        pltpu.make_async_copy(
            rows_v.at[b], out_hbm.at[pl.ds(bbase, GB)], sem_o.at[b]
        ).wait()

    fire(0, 0)

    def body(g, carry):
        b = lax.rem(g, NBUF)
        gn = g + 1
        bn = lax.rem(gn, NBUF)

        @pl.when(gn < NGROUP)
        def _():
            @pl.when(gn >= NBUF)
            def _():
                out_drain(bn)

            fire(gn, bn)

        drain_gather(b)
        out_start(g, b)
        return carry

    lax.fori_loop(0, NGROUP, body, 0, unroll=False)

    for b in range(min(NBUF, NGROUP)):
        out_drain(b)


@jax.jit
def kernel(token_ids, weights):
    idx = jnp.pad(
        token_ids.astype(jnp.int32), ((0, 0), (0, HPAD - HIST))
    ).reshape(BATCH * HPAD)
    mesh = plsc.VectorSubcoreMesh(core_axis_name="c", subcore_axis_name="s")
    out = pl.kernel(
        _gather_kernel,
        out_type=jax.ShapeDtypeStruct((BATCH, HIST, DIM), jnp.float32),
        mesh=mesh,
        scratch_types=[
            pltpu.VMEM((B_PER_W * HPAD,), jnp.int32),
            pltpu.VMEM((NBUF, GB, HIST, DIM), jnp.float32),
            pltpu.SemaphoreType.DMA((NBUF,)),
            pltpu.SemaphoreType.DMA((NBUF,)),
        ],
    )(weights, idx)
    return out
